# baseline (speedup 1.0000x reference)
import functools

import jax
import jax.numpy as jnp
from jax import lax
from jax.experimental import pallas as pl
from jax.experimental.pallas import tpu as pltpu
from jax.experimental.pallas import tpu_sc as plsc

N_NODES = 10000
N_EDGES = 320000
CH = 128

NC = 2
NS = 16
NW = NC * NS
EB = 128
NB = 80
CB = 16
N_PAD = 10240
E_PAD = NW * NB * EB
EPT = NB * EB
RPT = N_PAD // NS
ROWS = N_PAD // 8

_sc_mesh = plsc.VectorSubcoreMesh(core_axis_name="c", subcore_axis_name="s")


def _prep_body(ei_ref, src_ref, dst_ref):
    w = pl.program_id(0)
    e0 = w * EPT
    pos = e0 + lax.broadcasted_iota(jnp.int32, (1, 1, EPT), 2)
    real = pos < N_EDGES
    src_ref[...] = jnp.where(real, ei_ref[0:1, :][None], N_NODES)
    dst_ref[...] = jnp.where(
        real, ei_ref[1:2, :][None], N_NODES + pos % (N_PAD - N_NODES))


@functools.partial(
    pl.kernel,
    out_type=(jax.ShapeDtypeStruct((N_PAD,), jnp.float32),
              jax.ShapeDtypeStruct((N_PAD,), jnp.float32)),
    mesh=_sc_mesh,
    scratch_types=[
        pltpu.VMEM((NB, EB), jnp.int32),
        pltpu.VMEM((EB,), jnp.float32),
        pltpu.VMEM_SHARED((N_PAD,), jnp.float32),
        pltpu.SemaphoreType.DMA,
    ],
)
def _deg_kernel(dst_hbm, ones_hbm, zeros1_hbm, deg_out0, deg_out1, dst_v,
                ones_v, deg_sh, sem):
    cid = lax.axis_index("c")
    sid = lax.axis_index("s")
    wid = cid * NS + sid

    @pl.when(sid == 0)
    def _():
        pltpu.sync_copy(zeros1_hbm, deg_sh)

    pltpu.sync_copy(ones_hbm, ones_v)
    pltpu.sync_copy(dst_hbm.at[wid], dst_v)
    plsc.subcore_barrier()

    pltpu.async_copy(ones_v, deg_sh.at[dst_v.at[0]], sem, add=True)

    def body(i, _):
        @pl.when(i + 1 < NB)
        def _():
            pltpu.async_copy(ones_v, deg_sh.at[dst_v.at[i + 1]], sem, add=True)

        pltpu.make_async_copy(ones_v, deg_sh.at[dst_v.at[i]], sem).wait()
        return 0

    lax.fori_loop(0, NB, body, 0)
    plsc.subcore_barrier()

    @pl.when(jnp.logical_and(cid == 0, sid == 0))
    def _():
        pltpu.sync_copy(deg_sh, deg_out0)

    @pl.when(jnp.logical_and(cid == 1, sid == 0))
    def _():
        pltpu.sync_copy(deg_sh, deg_out1)


@functools.partial(
    pl.kernel,
    out_type=jax.ShapeDtypeStruct((NC, N_PAD, CH), jnp.float32),
    mesh=_sc_mesh,
    scratch_types=[
        pltpu.VMEM((CB, EB), jnp.int32),
        pltpu.VMEM((CB, EB), jnp.int32),
        pltpu.VMEM((2, EB, CH), jnp.float32),
        pltpu.VMEM_SHARED((N_PAD, CH), jnp.float32),
        pltpu.SemaphoreType.DMA,
        pltpu.SemaphoreType.DMA,
    ],
)
def _scatter_kernel(hp_hbm, src_hbm, dst_hbm, zeros_hbm, acc_out,
                    src_v, dst_v, rows_v, acc_sh, gsem, ssem):
    cid = lax.axis_index("c")
    sid = lax.axis_index("s")
    wid = cid * NS + sid

    pltpu.sync_copy(zeros_hbm.at[pl.ds(sid * RPT, RPT)],
                    acc_sh.at[pl.ds(sid * RPT, RPT)])
    plsc.subcore_barrier()

    def chunk(c, _):
        pltpu.sync_copy(src_hbm.at[wid, pl.ds(c * CB, CB)], src_v)
        pltpu.sync_copy(dst_hbm.at[wid, pl.ds(c * CB, CB)], dst_v)
        pltpu.async_copy(hp_hbm.at[src_v.at[0]], rows_v.at[0], gsem).wait()
        for j in range(CB):
            buf = j % 2
            nbuf = 1 - buf
            if j + 1 < CB:
                pltpu.async_copy(hp_hbm.at[src_v.at[j + 1]], rows_v.at[nbuf],
                                 gsem)
            pltpu.async_copy(rows_v.at[buf], acc_sh.at[dst_v.at[j]], ssem,
                             add=True).wait()
            if j + 1 < CB:
                pltpu.make_async_copy(hp_hbm.at[src_v.at[0]], rows_v.at[nbuf],
                                      gsem).wait()
        return 0

    lax.fori_loop(0, NB // CB, chunk, 0)
    plsc.subcore_barrier()
    pltpu.sync_copy(acc_sh.at[pl.ds(sid * RPT, RPT)],
                    acc_out.at[cid, pl.ds(sid * RPT, RPT)])


def _matmul_body(x_ref, w_ref, degbc_ref, hp_ref):
    dinv = lax.rsqrt(degbc_ref[...])
    h = jnp.dot(x_ref[...], w_ref[...], preferred_element_type=jnp.float32)
    hp_ref[...] = h * dinv


def _epilogue_body(accp_ref, hp_ref, degbc_ref, b_ref, out_ref):
    dinv = lax.rsqrt(degbc_ref[...])
    s = accp_ref[0] + accp_ref[1] + hp_ref[...]
    out_ref[...] = jnp.maximum(s * dinv + b_ref[...], 0.0)


def kernel(x, edge_index, W, b):
    ei = edge_index.astype(jnp.int32)

    src2, dst2 = pl.pallas_call(
        _prep_body,
        grid=(NW,),
        in_specs=[pl.BlockSpec((2, EPT), lambda i: (0, i))],
        out_specs=[
            pl.BlockSpec((1, 1, EPT), lambda i: (i, 0, 0)),
            pl.BlockSpec((1, 1, EPT), lambda i: (i, 0, 0)),
        ],
        out_shape=[
            jax.ShapeDtypeStruct((NW, 1, EPT), jnp.int32),
            jax.ShapeDtypeStruct((NW, 1, EPT), jnp.int32),
        ],
    )(ei)
    src3 = src2.reshape(NW, NB, EB)
    dst3 = dst2.reshape(NW, NB, EB)

    zeros1 = jnp.zeros((N_PAD,), jnp.float32)
    zeros_ch = jnp.zeros((N_PAD, CH), jnp.float32)
    ones1 = jnp.ones((EB,), jnp.float32)

    degp0, degp1 = _deg_kernel(dst3, ones1, zeros1)
    degbc = jnp.broadcast_to((degp0 + degp1 + 1.0)[:, None], (N_PAD, CH))

    hp = pl.pallas_call(
        _matmul_body,
        grid=(N_PAD // ROWS,),
        in_specs=[
            pl.BlockSpec((ROWS, CH), lambda i: (i, 0)),
            pl.BlockSpec((CH, CH), lambda i: (0, 0)),
            pl.BlockSpec((ROWS, CH), lambda i: (i, 0)),
        ],
        out_specs=pl.BlockSpec((ROWS, CH), lambda i: (i, 0)),
        out_shape=jax.ShapeDtypeStruct((N_PAD, CH), jnp.float32),
    )(x, W, degbc)

    accp = _scatter_kernel(hp, src3, dst3, zeros_ch)

    out = pl.pallas_call(
        _epilogue_body,
        grid=(N_PAD // ROWS,),
        in_specs=[
            pl.BlockSpec((NC, ROWS, CH), lambda i: (0, i, 0)),
            pl.BlockSpec((ROWS, CH), lambda i: (i, 0)),
            pl.BlockSpec((ROWS, CH), lambda i: (i, 0)),
            pl.BlockSpec((CH,), lambda i: (0,)),
        ],
        out_specs=pl.BlockSpec((ROWS, CH), lambda i: (i, 0)),
        out_shape=jax.ShapeDtypeStruct((N_NODES, CH), jnp.float32),
    )(accp, hp, degbc, b)

    return out


# speedup vs baseline: 3.2697x; 3.2697x over previous
import functools

import jax
import jax.numpy as jnp
from jax import lax
from jax.experimental import pallas as pl
from jax.experimental.pallas import tpu as pltpu
from jax.experimental.pallas import tpu_sc as plsc

N_NODES = 10000
N_EDGES = 320000
CH = 128

NC = 2
NS = 16
NW = NC * NS
EB = 128
NB = 80
CB = 16
N_PAD = 10240
E_PAD = NW * NB * EB
EPT = NB * EB
RPT = N_PAD // NS
ROWS = N_PAD // 8

_sc_mesh = plsc.VectorSubcoreMesh(core_axis_name="c", subcore_axis_name="s")


_PREP_B = E_PAD // 4


def _prep_body(ei_ref, src_ref, dst_ref):
    w = pl.program_id(0)
    pos = w * _PREP_B + lax.broadcasted_iota(jnp.int32, (1, _PREP_B), 1)
    real = pos < N_EDGES
    fill = N_NODES + pos % (N_PAD - N_NODES)
    src_ref[...] = jnp.where(real, ei_ref[0:1, :], fill)
    dst_ref[...] = jnp.where(real, ei_ref[1:2, :], fill)


@functools.partial(
    pl.kernel,
    out_type=(jax.ShapeDtypeStruct((N_PAD,), jnp.float32),
              jax.ShapeDtypeStruct((N_PAD,), jnp.float32)),
    mesh=_sc_mesh,
    scratch_types=[
        pltpu.VMEM((NB, EB), jnp.int32),
        pltpu.VMEM((EB,), jnp.float32),
        pltpu.VMEM_SHARED((N_PAD,), jnp.float32),
        pltpu.SemaphoreType.DMA,
    ],
)
def _deg_kernel(dst_hbm, ones_hbm, zeros1_hbm, deg_out0, deg_out1, dst_v,
                ones_v, deg_sh, sem):
    cid = lax.axis_index("c")
    sid = lax.axis_index("s")
    wid = cid * NS + sid

    @pl.when(sid == 0)
    def _():
        pltpu.sync_copy(zeros1_hbm, deg_sh)

    pltpu.sync_copy(ones_hbm, ones_v)
    pltpu.sync_copy(dst_hbm.at[wid], dst_v)
    plsc.subcore_barrier()

    pltpu.async_copy(ones_v, deg_sh.at[dst_v.at[0]], sem, add=True)

    def body(i, _):
        @pl.when(i + 1 < NB)
        def _():
            pltpu.async_copy(ones_v, deg_sh.at[dst_v.at[i + 1]], sem, add=True)

        pltpu.make_async_copy(ones_v, deg_sh.at[dst_v.at[i]], sem).wait()
        return 0

    lax.fori_loop(0, NB, body, 0)
    plsc.subcore_barrier()

    @pl.when(jnp.logical_and(cid == 0, sid == 0))
    def _():
        pltpu.sync_copy(deg_sh, deg_out0)

    @pl.when(jnp.logical_and(cid == 1, sid == 0))
    def _():
        pltpu.sync_copy(deg_sh, deg_out1)


@functools.partial(
    pl.kernel,
    out_type=jax.ShapeDtypeStruct((NC, N_PAD, CH), jnp.float32),
    mesh=_sc_mesh,
    scratch_types=[
        pltpu.VMEM((CB, EB), jnp.int32),
        pltpu.VMEM((CB, EB), jnp.int32),
        pltpu.VMEM((2, EB, CH), jnp.float32),
        pltpu.VMEM_SHARED((N_PAD, CH), jnp.float32),
        pltpu.SemaphoreType.DMA,
        pltpu.SemaphoreType.DMA,
    ],
)
def _scatter_kernel(hp_hbm, src_hbm, dst_hbm, zeros_hbm, acc_out,
                    src_v, dst_v, rows_v, acc_sh, gsem, ssem):
    cid = lax.axis_index("c")
    sid = lax.axis_index("s")
    wid = cid * NS + sid

    pltpu.sync_copy(zeros_hbm.at[pl.ds(sid * RPT, RPT)],
                    acc_sh.at[pl.ds(sid * RPT, RPT)])
    plsc.subcore_barrier()

    def chunk(c, _):
        pltpu.sync_copy(src_hbm.at[wid, pl.ds(c * CB, CB)], src_v)
        pltpu.sync_copy(dst_hbm.at[wid, pl.ds(c * CB, CB)], dst_v)
        pltpu.async_copy(hp_hbm.at[src_v.at[0]], rows_v.at[0], gsem).wait()
        for j in range(CB):
            buf = j % 2
            nbuf = 1 - buf
            if j + 1 < CB:
                pltpu.async_copy(hp_hbm.at[src_v.at[j + 1]], rows_v.at[nbuf],
                                 gsem)
            pltpu.async_copy(rows_v.at[buf], acc_sh.at[dst_v.at[j]], ssem,
                             add=True).wait()
            if j + 1 < CB:
                pltpu.make_async_copy(hp_hbm.at[src_v.at[0]], rows_v.at[nbuf],
                                      gsem).wait()
        return 0

    lax.fori_loop(0, NB // CB, chunk, 0)
    plsc.subcore_barrier()
    pltpu.sync_copy(acc_sh.at[pl.ds(sid * RPT, RPT)],
                    acc_out.at[cid, pl.ds(sid * RPT, RPT)])


def _matmul_body(x_ref, w_ref, degbc_ref, hp_ref):
    dinv = lax.rsqrt(degbc_ref[...])
    h = jnp.dot(x_ref[...], w_ref[...], preferred_element_type=jnp.float32)
    hp_ref[...] = h * dinv


def _epilogue_body(accp_ref, hp_ref, degbc_ref, b_ref, out_ref):
    dinv = lax.rsqrt(degbc_ref[...])
    s = accp_ref[0] + accp_ref[1] + hp_ref[...]
    out_ref[...] = jnp.maximum(s * dinv + b_ref[...], 0.0)


def kernel(x, edge_index, W, b):
    ei = edge_index.astype(jnp.int32)

    src2, dst2 = pl.pallas_call(
        _prep_body,
        grid=(4,),
        in_specs=[pl.BlockSpec((2, _PREP_B), lambda i: (0, i))],
        out_specs=[
            pl.BlockSpec((1, _PREP_B), lambda i: (0, i)),
            pl.BlockSpec((1, _PREP_B), lambda i: (0, i)),
        ],
        out_shape=[
            jax.ShapeDtypeStruct((1, E_PAD), jnp.int32),
            jax.ShapeDtypeStruct((1, E_PAD), jnp.int32),
        ],
    )(ei)
    src3 = src2.reshape(NW, NB, EB)
    dst3 = dst2.reshape(NW, NB, EB)

    zeros1 = jnp.zeros((N_PAD,), jnp.float32)
    zeros_ch = jnp.zeros((N_PAD, CH), jnp.float32)
    ones1 = jnp.ones((EB,), jnp.float32)

    degp0, degp1 = _deg_kernel(dst3, ones1, zeros1)
    degbc = jnp.broadcast_to((degp0 + degp1 + 1.0)[:, None], (N_PAD, CH))

    hp = pl.pallas_call(
        _matmul_body,
        grid=(N_PAD // ROWS,),
        in_specs=[
            pl.BlockSpec((ROWS, CH), lambda i: (i, 0)),
            pl.BlockSpec((CH, CH), lambda i: (0, 0)),
            pl.BlockSpec((ROWS, CH), lambda i: (i, 0)),
        ],
        out_specs=pl.BlockSpec((ROWS, CH), lambda i: (i, 0)),
        out_shape=jax.ShapeDtypeStruct((N_PAD, CH), jnp.float32),
    )(x, W, degbc)

    accp = _scatter_kernel(hp, src3, dst3, zeros_ch)

    out = pl.pallas_call(
        _epilogue_body,
        grid=(N_PAD // ROWS,),
        in_specs=[
            pl.BlockSpec((NC, ROWS, CH), lambda i: (0, i, 0)),
            pl.BlockSpec((ROWS, CH), lambda i: (i, 0)),
            pl.BlockSpec((ROWS, CH), lambda i: (i, 0)),
            pl.BlockSpec((CH,), lambda i: (0,)),
        ],
        out_specs=pl.BlockSpec((ROWS, CH), lambda i: (i, 0)),
        out_shape=jax.ShapeDtypeStruct((N_NODES, CH), jnp.float32),
    )(accp, hp, degbc, b)

    return out


# speedup vs baseline: 3.3605x; 1.0278x over previous
import functools

import jax
import jax.numpy as jnp
from jax import lax
from jax.experimental import pallas as pl
from jax.experimental.pallas import tpu as pltpu
from jax.experimental.pallas import tpu_sc as plsc

N_NODES = 10000
N_EDGES = 320000
CH = 128

NC = 2
NS = 16
NW = NC * NS
EB = 128
NB = 80
CB = 16
N_PAD = 10240
E_PAD = NW * NB * EB
EPT = NB * EB
RPT = N_PAD // NS
ROWS = N_PAD // 8

_sc_mesh = plsc.VectorSubcoreMesh(core_axis_name="c", subcore_axis_name="s")


_PREP_B = E_PAD // 4


def _prep_body(ei_ref, src_ref, dst_ref):
    w = pl.program_id(0)
    pos = w * _PREP_B + lax.broadcasted_iota(jnp.int32, (1, _PREP_B), 1)
    real = pos < N_EDGES
    fill = N_NODES + pos % (N_PAD - N_NODES)
    src_ref[...] = jnp.where(real, ei_ref[0:1, :], fill)
    dst_ref[...] = jnp.where(real, ei_ref[1:2, :], fill)


@functools.partial(
    pl.kernel,
    out_type=(jax.ShapeDtypeStruct((N_PAD,), jnp.float32),
              jax.ShapeDtypeStruct((N_PAD,), jnp.float32)),
    mesh=_sc_mesh,
    scratch_types=[
        pltpu.VMEM((NB, EB), jnp.int32),
        pltpu.VMEM((EB,), jnp.float32),
        pltpu.VMEM_SHARED((N_PAD,), jnp.float32),
        pltpu.SemaphoreType.DMA,
    ],
)
def _deg_kernel(dst_hbm, ones_hbm, zeros1_hbm, deg_out0, deg_out1, dst_v,
                ones_v, deg_sh, sem):
    cid = lax.axis_index("c")
    sid = lax.axis_index("s")
    wid = cid * NS + sid

    @pl.when(sid == 0)
    def _():
        pltpu.sync_copy(zeros1_hbm, deg_sh)

    pltpu.sync_copy(ones_hbm, ones_v)
    pltpu.sync_copy(dst_hbm.at[wid], dst_v)
    plsc.subcore_barrier()

    pltpu.async_copy(ones_v, deg_sh.at[dst_v.at[0]], sem, add=True)

    def body(i, _):
        @pl.when(i + 1 < NB)
        def _():
            pltpu.async_copy(ones_v, deg_sh.at[dst_v.at[i + 1]], sem, add=True)

        pltpu.make_async_copy(ones_v, deg_sh.at[dst_v.at[i]], sem).wait()
        return 0

    lax.fori_loop(0, NB, body, 0)
    plsc.subcore_barrier()

    @pl.when(jnp.logical_and(cid == 0, sid == 0))
    def _():
        pltpu.sync_copy(deg_sh, deg_out0)

    @pl.when(jnp.logical_and(cid == 1, sid == 0))
    def _():
        pltpu.sync_copy(deg_sh, deg_out1)


@functools.partial(
    pl.kernel,
    out_type=jax.ShapeDtypeStruct((NC, N_PAD, CH), jnp.float32),
    mesh=_sc_mesh,
    scratch_types=[
        pltpu.VMEM((2, CB, EB), jnp.int32),
        pltpu.VMEM((2, CB, EB), jnp.int32),
        pltpu.VMEM((2, EB, CH), jnp.float32),
        pltpu.VMEM_SHARED((N_PAD, CH), jnp.float32),
        pltpu.SemaphoreType.DMA,
        pltpu.SemaphoreType.DMA,
        pltpu.SemaphoreType.DMA,
    ],
)
def _scatter_kernel(hp_hbm, src_hbm, dst_hbm, zeros_hbm, acc_out,
                    src_v, dst_v, rows_v, acc_sh, gsem, ssem, isem):
    cid = lax.axis_index("c")
    sid = lax.axis_index("s")
    wid = cid * NS + sid
    nch = NB // CB

    zdesc = pltpu.async_copy(zeros_hbm.at[pl.ds(sid * RPT, RPT)],
                             acc_sh.at[pl.ds(sid * RPT, RPT)], ssem)
    pltpu.sync_copy(src_hbm.at[wid, pl.ds(0, CB)], src_v.at[0])
    pltpu.sync_copy(dst_hbm.at[wid, pl.ds(0, CB)], dst_v.at[0])
    zdesc.wait()
    plsc.subcore_barrier()

    def chunk(c, _):
        slot = lax.rem(c, 2)
        nslot = lax.rem(c + 1, 2)
        srcs = src_v.at[slot]
        dsts = dst_v.at[slot]

        @pl.when(c + 1 < nch)
        def _():
            pltpu.async_copy(src_hbm.at[wid, pl.ds((c + 1) * CB, CB)],
                             src_v.at[nslot], isem)
            pltpu.async_copy(dst_hbm.at[wid, pl.ds((c + 1) * CB, CB)],
                             dst_v.at[nslot], isem)

        pltpu.async_copy(hp_hbm.at[srcs.at[0]], rows_v.at[0], gsem).wait()
        for j in range(CB):
            buf = j % 2
            nbuf = 1 - buf
            if j + 1 < CB:
                pltpu.async_copy(hp_hbm.at[srcs.at[j + 1]], rows_v.at[nbuf],
                                 gsem)
            pltpu.async_copy(rows_v.at[buf], acc_sh.at[dsts.at[j]], ssem,
                             add=True).wait()
            if j + 1 < CB:
                pltpu.make_async_copy(hp_hbm.at[srcs.at[0]], rows_v.at[nbuf],
                                      gsem).wait()

        @pl.when(c + 1 < nch)
        def _():
            pltpu.make_async_copy(src_hbm.at[wid, pl.ds(0, CB)],
                                  src_v.at[nslot], isem).wait()
            pltpu.make_async_copy(dst_hbm.at[wid, pl.ds(0, CB)],
                                  dst_v.at[nslot], isem).wait()
        return 0

    lax.fori_loop(0, NB // CB, chunk, 0)
    plsc.subcore_barrier()
    pltpu.sync_copy(acc_sh.at[pl.ds(sid * RPT, RPT)],
                    acc_out.at[cid, pl.ds(sid * RPT, RPT)])


def _matmul_body(x_ref, w_ref, h_ref):
    h_ref[...] = jnp.dot(x_ref[...], w_ref[...],
                         preferred_element_type=jnp.float32)


def _scale_body(h_ref, degbc_ref, hp_ref):
    hp_ref[...] = h_ref[...] * lax.rsqrt(degbc_ref[...])


def _epilogue_body(accp_ref, hp_ref, degbc_ref, b_ref, out_ref):
    dinv = lax.rsqrt(degbc_ref[...])
    s = accp_ref[0] + accp_ref[1] + hp_ref[...]
    out_ref[...] = jnp.maximum(s * dinv + b_ref[...], 0.0)


def kernel(x, edge_index, W, b):
    ei = edge_index.astype(jnp.int32)

    src2, dst2 = pl.pallas_call(
        _prep_body,
        grid=(4,),
        in_specs=[pl.BlockSpec((2, _PREP_B), lambda i: (0, i))],
        out_specs=[
            pl.BlockSpec((1, _PREP_B), lambda i: (0, i)),
            pl.BlockSpec((1, _PREP_B), lambda i: (0, i)),
        ],
        out_shape=[
            jax.ShapeDtypeStruct((1, E_PAD), jnp.int32),
            jax.ShapeDtypeStruct((1, E_PAD), jnp.int32),
        ],
    )(ei)
    src3 = src2.reshape(NW, NB, EB)
    dst3 = dst2.reshape(NW, NB, EB)

    zeros1 = jnp.zeros((N_PAD,), jnp.float32)
    zeros_ch = jnp.zeros((N_PAD, CH), jnp.float32)
    ones1 = jnp.ones((EB,), jnp.float32)

    h = pl.pallas_call(
        _matmul_body,
        grid=(N_PAD // ROWS,),
        in_specs=[
            pl.BlockSpec((ROWS, CH), lambda i: (i, 0)),
            pl.BlockSpec((CH, CH), lambda i: (0, 0)),
        ],
        out_specs=pl.BlockSpec((ROWS, CH), lambda i: (i, 0)),
        out_shape=jax.ShapeDtypeStruct((N_PAD, CH), jnp.float32),
    )(x, W)

    degp0, degp1 = _deg_kernel(dst3, ones1, zeros1)
    degbc = jnp.broadcast_to((degp0 + degp1 + 1.0)[:, None], (N_PAD, CH))

    hp = pl.pallas_call(
        _scale_body,
        grid=(N_PAD // ROWS,),
        in_specs=[
            pl.BlockSpec((ROWS, CH), lambda i: (i, 0)),
            pl.BlockSpec((ROWS, CH), lambda i: (i, 0)),
        ],
        out_specs=pl.BlockSpec((ROWS, CH), lambda i: (i, 0)),
        out_shape=jax.ShapeDtypeStruct((N_PAD, CH), jnp.float32),
    )(h, degbc)

    accp = _scatter_kernel(hp, src3, dst3, zeros_ch)

    out = pl.pallas_call(
        _epilogue_body,
        grid=(N_PAD // ROWS,),
        in_specs=[
            pl.BlockSpec((NC, ROWS, CH), lambda i: (0, i, 0)),
            pl.BlockSpec((ROWS, CH), lambda i: (i, 0)),
            pl.BlockSpec((ROWS, CH), lambda i: (i, 0)),
            pl.BlockSpec((CH,), lambda i: (0,)),
        ],
        out_specs=pl.BlockSpec((ROWS, CH), lambda i: (i, 0)),
        out_shape=jax.ShapeDtypeStruct((N_NODES, CH), jnp.float32),
    )(accp, hp, degbc, b)

    return out


# speedup vs baseline: 3.3933x; 1.0097x over previous
import functools

import jax
import jax.numpy as jnp
from jax import lax
from jax.experimental import pallas as pl
from jax.experimental.pallas import tpu as pltpu
from jax.experimental.pallas import tpu_sc as plsc

N_NODES = 10000
N_EDGES = 320000
CH = 128

NC = 2
NS = 16
NW = NC * NS
EB = 128
NB = 80
CB = 16
N_PAD = 10240
E_PAD = NW * NB * EB
EPT = NB * EB
RPT = N_PAD // NS
ROWS = N_PAD // 4

_sc_mesh = plsc.VectorSubcoreMesh(core_axis_name="c", subcore_axis_name="s")


_PREP_B = E_PAD // 2


def _prep_body(ei_ref, src_ref, dst_ref):
    w = pl.program_id(0)
    pos = w * _PREP_B + lax.broadcasted_iota(jnp.int32, (1, _PREP_B), 1)
    real = pos < N_EDGES
    fill = N_NODES + pos % (N_PAD - N_NODES)
    src_ref[...] = jnp.where(real, ei_ref[0:1, :], fill)
    dst_ref[...] = jnp.where(real, ei_ref[1:2, :], fill)


@functools.partial(
    pl.kernel,
    out_type=(jax.ShapeDtypeStruct((N_PAD,), jnp.float32),
              jax.ShapeDtypeStruct((N_PAD,), jnp.float32)),
    mesh=_sc_mesh,
    scratch_types=[
        pltpu.VMEM((NB, EB), jnp.int32),
        pltpu.VMEM((EB,), jnp.float32),
        pltpu.VMEM_SHARED((N_PAD,), jnp.float32),
        pltpu.SemaphoreType.DMA,
    ],
)
def _deg_kernel(dst_hbm, ones_hbm, zeros1_hbm, deg_out0, deg_out1, dst_v,
                ones_v, deg_sh, sem):
    cid = lax.axis_index("c")
    sid = lax.axis_index("s")
    wid = cid * NS + sid

    @pl.when(sid == 0)
    def _():
        pltpu.sync_copy(zeros1_hbm, deg_sh)

    pltpu.sync_copy(ones_hbm, ones_v)
    pltpu.sync_copy(dst_hbm.at[wid], dst_v)
    plsc.subcore_barrier()

    pltpu.async_copy(ones_v, deg_sh.at[dst_v.at[0]], sem, add=True)

    def body(i, _):
        @pl.when(i + 1 < NB)
        def _():
            pltpu.async_copy(ones_v, deg_sh.at[dst_v.at[i + 1]], sem, add=True)

        pltpu.make_async_copy(ones_v, deg_sh.at[dst_v.at[i]], sem).wait()
        return 0

    lax.fori_loop(0, NB, body, 0)
    plsc.subcore_barrier()

    @pl.when(jnp.logical_and(cid == 0, sid == 0))
    def _():
        pltpu.sync_copy(deg_sh, deg_out0)

    @pl.when(jnp.logical_and(cid == 1, sid == 0))
    def _():
        pltpu.sync_copy(deg_sh, deg_out1)


@functools.partial(
    pl.kernel,
    out_type=jax.ShapeDtypeStruct((NC, N_PAD, CH), jnp.float32),
    mesh=_sc_mesh,
    scratch_types=[
        pltpu.VMEM((2, CB, EB), jnp.int32),
        pltpu.VMEM((2, CB, EB), jnp.int32),
        pltpu.VMEM((2, EB, CH), jnp.float32),
        pltpu.VMEM_SHARED((N_PAD, CH), jnp.float32),
        pltpu.SemaphoreType.DMA,
        pltpu.SemaphoreType.DMA,
        pltpu.SemaphoreType.DMA,
    ],
)
def _scatter_kernel(hp_hbm, src_hbm, dst_hbm, zeros_hbm, acc_out,
                    src_v, dst_v, rows_v, acc_sh, gsem, ssem, isem):
    cid = lax.axis_index("c")
    sid = lax.axis_index("s")
    wid = cid * NS + sid
    nch = NB // CB

    zdesc = pltpu.async_copy(zeros_hbm.at[pl.ds(sid * RPT, RPT)],
                             acc_sh.at[pl.ds(sid * RPT, RPT)], ssem)
    pltpu.sync_copy(src_hbm.at[wid, pl.ds(0, CB)], src_v.at[0])
    pltpu.sync_copy(dst_hbm.at[wid, pl.ds(0, CB)], dst_v.at[0])
    zdesc.wait()
    plsc.subcore_barrier()

    def chunk(c, _):
        slot = lax.rem(c, 2)
        nslot = lax.rem(c + 1, 2)
        srcs = src_v.at[slot]
        dsts = dst_v.at[slot]

        @pl.when(c + 1 < nch)
        def _():
            pltpu.async_copy(src_hbm.at[wid, pl.ds((c + 1) * CB, CB)],
                             src_v.at[nslot], isem)
            pltpu.async_copy(dst_hbm.at[wid, pl.ds((c + 1) * CB, CB)],
                             dst_v.at[nslot], isem)

        pltpu.async_copy(hp_hbm.at[srcs.at[0]], rows_v.at[0], gsem).wait()
        for j in range(CB):
            buf = j % 2
            nbuf = 1 - buf
            if j + 1 < CB:
                pltpu.async_copy(hp_hbm.at[srcs.at[j + 1]], rows_v.at[nbuf],
                                 gsem)
            pltpu.async_copy(rows_v.at[buf], acc_sh.at[dsts.at[j]], ssem,
                             add=True).wait()
            if j + 1 < CB:
                pltpu.make_async_copy(hp_hbm.at[srcs.at[0]], rows_v.at[nbuf],
                                      gsem).wait()

        @pl.when(c + 1 < nch)
        def _():
            pltpu.make_async_copy(src_hbm.at[wid, pl.ds(0, CB)],
                                  src_v.at[nslot], isem).wait()
            pltpu.make_async_copy(dst_hbm.at[wid, pl.ds(0, CB)],
                                  dst_v.at[nslot], isem).wait()
        return 0

    lax.fori_loop(0, NB // CB, chunk, 0)
    plsc.subcore_barrier()
    pltpu.sync_copy(acc_sh.at[pl.ds(sid * RPT, RPT)],
                    acc_out.at[cid, pl.ds(sid * RPT, RPT)])


def _matmul_body(x_ref, w_ref, h_ref):
    h_ref[...] = jnp.dot(x_ref[...], w_ref[...],
                         preferred_element_type=jnp.float32)


def _scale_body(h_ref, degbc_ref, hp_ref):
    hp_ref[...] = h_ref[...] * lax.rsqrt(degbc_ref[...])


def _epilogue_body(accp_ref, hp_ref, degbc_ref, b_ref, out_ref):
    dinv = lax.rsqrt(degbc_ref[...])
    s = accp_ref[0] + accp_ref[1] + hp_ref[...]
    out_ref[...] = jnp.maximum(s * dinv + b_ref[...], 0.0)


def kernel(x, edge_index, W, b):
    ei = edge_index.astype(jnp.int32)

    src2, dst2 = pl.pallas_call(
        _prep_body,
        grid=(2,),
        in_specs=[pl.BlockSpec((2, _PREP_B), lambda i: (0, i))],
        out_specs=[
            pl.BlockSpec((1, _PREP_B), lambda i: (0, i)),
            pl.BlockSpec((1, _PREP_B), lambda i: (0, i)),
        ],
        out_shape=[
            jax.ShapeDtypeStruct((1, E_PAD), jnp.int32),
            jax.ShapeDtypeStruct((1, E_PAD), jnp.int32),
        ],
    )(ei)
    src3 = src2.reshape(NW, NB, EB)
    dst3 = dst2.reshape(NW, NB, EB)

    zeros1 = jnp.zeros((N_PAD,), jnp.float32)
    zeros_ch = jnp.zeros((N_PAD, CH), jnp.float32)
    ones1 = jnp.ones((EB,), jnp.float32)

    h = pl.pallas_call(
        _matmul_body,
        grid=(N_PAD // ROWS,),
        in_specs=[
            pl.BlockSpec((ROWS, CH), lambda i: (i, 0)),
            pl.BlockSpec((CH, CH), lambda i: (0, 0)),
        ],
        out_specs=pl.BlockSpec((ROWS, CH), lambda i: (i, 0)),
        out_shape=jax.ShapeDtypeStruct((N_PAD, CH), jnp.float32),
    )(x, W)

    degp0, degp1 = _deg_kernel(dst3, ones1, zeros1)
    degbc = jnp.broadcast_to((degp0 + degp1 + 1.0)[:, None], (N_PAD, CH))

    hp = pl.pallas_call(
        _scale_body,
        grid=(N_PAD // ROWS,),
        in_specs=[
            pl.BlockSpec((ROWS, CH), lambda i: (i, 0)),
            pl.BlockSpec((ROWS, CH), lambda i: (i, 0)),
        ],
        out_specs=pl.BlockSpec((ROWS, CH), lambda i: (i, 0)),
        out_shape=jax.ShapeDtypeStruct((N_PAD, CH), jnp.float32),
    )(h, degbc)

    accp = _scatter_kernel(hp, src3, dst3, zeros_ch)

    out = pl.pallas_call(
        _epilogue_body,
        grid=(N_PAD // ROWS,),
        in_specs=[
            pl.BlockSpec((NC, ROWS, CH), lambda i: (0, i, 0)),
            pl.BlockSpec((ROWS, CH), lambda i: (i, 0)),
            pl.BlockSpec((ROWS, CH), lambda i: (i, 0)),
            pl.BlockSpec((CH,), lambda i: (0,)),
        ],
        out_specs=pl.BlockSpec((ROWS, CH), lambda i: (i, 0)),
        out_shape=jax.ShapeDtypeStruct((N_NODES, CH), jnp.float32),
    )(accp, hp, degbc, b)

    return out
